# baseline (speedup 1.0000x reference)
import functools

import jax
import jax.numpy as jnp
from jax import lax
from jax.experimental import pallas as pl
from jax.experimental.pallas import tpu as pltpu
from jax.experimental.pallas import tpu_sc as plsc

GAMMA = 1.0
CHUNK = 128


def _vec_sqrt(x):
    i = lax.bitcast_convert_type(x, jnp.int32)
    y = lax.bitcast_convert_type(
        jnp.int32(0x5F3759DF) - lax.shift_right_arithmetic(i, 1), jnp.float32
    )
    for _ in range(3):
        y = y * (1.5 - 0.5 * x * y * y)
    return x * y


def _sc_partials(idx2d, entity_embedding, relation_embedding, num_cores,
                 num_subcores, rows_per_worker):
    n_chunks = rows_per_worker // CHUNK
    n_groups = rows_per_worker // 16
    mesh = plsc.VectorSubcoreMesh(core_axis_name="c", subcore_axis_name="s")
    nw = num_cores * num_subcores
    d = entity_embedding.shape[1]

    @functools.partial(
        pl.kernel,
        out_type=jax.ShapeDtypeStruct((nw, 16), jnp.float32),
        mesh=mesh,
        scratch_types=[
            pltpu.VMEM((n_chunks, CHUNK), jnp.int32),
            pltpu.VMEM((n_chunks, CHUNK), jnp.int32),
            pltpu.VMEM((n_chunks, CHUNK), jnp.int32),
            pltpu.VMEM((rows_per_worker, d), jnp.float32),
            pltpu.VMEM((rows_per_worker, d), jnp.float32),
            pltpu.VMEM((rows_per_worker, d), jnp.float32),
            pltpu.VMEM((rows_per_worker,), jnp.float32),
            pltpu.VMEM((16,), jnp.float32),
            pltpu.SemaphoreType.DMA,
        ],
        compiler_params=pltpu.CompilerParams(
            needs_layout_passes=False, use_tc_tiling_on_sc=False
        ),
    )
    def sc_kernel(ph, pr, pt, nh, nr, nt, ent, rel, out_hbm,
                  hidx, ridx, tidx, hbuf, rbuf, tbuf, norms, accv, sem):
        wid = lax.axis_index("s") * num_cores + lax.axis_index("c")
        rbase = wid * n_chunks
        iota = lax.iota(jnp.int32, 16)

        def fetch(side_h, side_r, side_t):
            pltpu.sync_copy(side_h.at[pl.ds(rbase, n_chunks)], hidx)
            pltpu.sync_copy(side_r.at[pl.ds(rbase, n_chunks)], ridx)
            pltpu.sync_copy(side_t.at[pl.ds(rbase, n_chunks)], tidx)
            descs = []
            for c in range(n_chunks):
                dst = pl.ds(c * CHUNK, CHUNK)
                descs.append(pltpu.async_copy(ent.at[hidx.at[c]], hbuf.at[dst], sem))
                descs.append(pltpu.async_copy(rel.at[ridx.at[c]], rbuf.at[dst], sem))
                descs.append(pltpu.async_copy(ent.at[tidx.at[c]], tbuf.at[dst], sem))
            for de in descs:
                de.wait()

        def group_ssq(g):
            def row_body(r, ssqv):
                row = g * 16 + r
                acc = jnp.zeros((16,), jnp.float32)
                for k in range(d // 16):
                    sl = pl.ds(k * 16, 16)
                    a = hbuf[row, sl]
                    b = rbuf[row, sl]
                    t = tbuf[row, sl]
                    dv = a + b - t
                    acc = acc + dv * dv
                return jnp.where(iota == r, jnp.sum(acc), ssqv)

            return lax.fori_loop(0, 16, row_body,
                                 jnp.zeros((16,), jnp.float32))

        fetch(ph, pr, pt)

        def pos_group(g, carry):
            norms[pl.ds(g * 16, 16)] = _vec_sqrt(group_ssq(g))
            return carry

        lax.fori_loop(0, n_groups, pos_group, 0)

        fetch(nh, nr, nt)

        def neg_group(g, acc):
            nn = _vec_sqrt(group_ssq(g))
            pn = norms[pl.ds(g * 16, 16)]
            return acc + jnp.maximum(GAMMA + pn - nn, 0.0)

        total = lax.fori_loop(0, n_groups, neg_group,
                              jnp.zeros((16,), jnp.float32))
        accv[...] = total
        pltpu.sync_copy(accv, out_hbm.at[wid])

    return sc_kernel(*idx2d, entity_embedding, relation_embedding)


def _tc_reduce(partials):
    def body(x_ref, o_ref):
        o_ref[...] = jnp.sum(x_ref[...]).reshape(1, 1)

    return pl.pallas_call(
        body, out_shape=jax.ShapeDtypeStruct((1, 1), jnp.float32)
    )(partials)


def kernel(pos_head, pos_relation, pos_tail, neg_head, neg_relation,
           neg_tail, entity_embedding, relation_embedding):
    info = plsc.get_sparse_core_info()
    nw = info.num_cores * info.num_subcores
    batch = pos_head.shape[0]
    rows_per_worker = batch // nw
    idx2d = [
        a.astype(jnp.int32).reshape(batch // CHUNK, CHUNK)
        for a in (pos_head, pos_relation, pos_tail,
                  neg_head, neg_relation, neg_tail)
    ]
    partials = _sc_partials(idx2d, entity_embedding, relation_embedding,
                            info.num_cores, info.num_subcores,
                            rows_per_worker)
    return _tc_reduce(partials)[0, 0]
